# baseline (speedup 1.0000x reference)
import jax
import jax.numpy as jnp
from jax.experimental import pallas as pl

_N = 50000
_E = 800000
_S = 4
_L_MAX = 2
_N_MAX = 8
_R_CUT = 5.0
_BLK = 1000
_W = _N_MAX * (1 + 3 + 5)


def _expand_kernel(pc_ref, pn_ref, out_ref):
    v = pn_ref[...] - pc_ref[...]
    r = jnp.sqrt(jnp.sum(v * v, axis=-1) + 1e-12)
    d = v / r[:, None]
    fc = jnp.where(r < _R_CUT, 0.5 * (jnp.cos(jnp.pi * r / _R_CUT) + 1.0), 0.0)
    n = jnp.arange(1, _N_MAX + 1, dtype=jnp.int32).astype(jnp.float32)
    rb = jnp.sin(jnp.pi * n[None, :] * r[:, None] / _R_CUT) / r[:, None]
    rb = rb * fc[:, None]
    dx, dy, dz = d[:, 0], d[:, 1], d[:, 2]
    c0 = 0.28209479177387814
    c1 = 0.4886025119029199
    c2a = 1.0925484305920792
    c2b = 0.31539156525252005
    c2c = 0.5462742152960396
    ms = [
        jnp.full_like(dx, c0),
        c1 * dy, c1 * dz, c1 * dx,
        c2a * dx * dy,
        c2a * dy * dz,
        c2b * (3.0 * dz * dz - 1.0),
        c2a * dx * dz,
        c2c * (dx * dx - dy * dy),
    ]
    out_ref[...] = jnp.concatenate([m[:, None] * rb for m in ms], axis=-1)


def kernel(positions, edge_index, species):
    centers = edge_index[0].astype(jnp.int32)
    neighbors = edge_index[1].astype(jnp.int32)
    pc = positions[centers]
    pn = positions[neighbors]
    feat = pl.pallas_call(
        _expand_kernel,
        grid=(_E // _BLK,),
        in_specs=[
            pl.BlockSpec((_BLK, 3), lambda i: (i, 0)),
            pl.BlockSpec((_BLK, 3), lambda i: (i, 0)),
        ],
        out_specs=pl.BlockSpec((_BLK, _W), lambda i: (i, 0)),
        out_shape=jax.ShapeDtypeStruct((_E, _W), jnp.float32),
    )(pc, pn)
    seg = centers * _S + species[neighbors].astype(jnp.int32)
    summed = jax.ops.segment_sum(feat, seg, num_segments=_N * _S)
    outs = []
    off = 0
    for l in range(_L_MAX + 1):
        w = (2 * l + 1) * _N_MAX
        outs.append(summed[:, off:off + w].reshape(_N, _S * w))
        off += w
    return jnp.concatenate(outs, axis=-1)


# speedup vs baseline: 1.0027x; 1.0027x over previous
import jax
import jax.numpy as jnp
from jax.experimental import pallas as pl

_N = 50000
_E = 800000
_S = 4
_L_MAX = 2
_N_MAX = 8
_R_CUT = 5.0
_BLK = 4000
_W = _N_MAX * (1 + 3 + 5)


def _expand_kernel(pc_ref, pn_ref, out_ref):
    v = pn_ref[...] - pc_ref[...]
    r = jnp.sqrt(jnp.sum(v * v, axis=-1) + 1e-12)
    d = v / r[:, None]
    fc = jnp.where(r < _R_CUT, 0.5 * (jnp.cos(jnp.pi * r / _R_CUT) + 1.0), 0.0)
    n = jnp.arange(1, _N_MAX + 1, dtype=jnp.int32).astype(jnp.float32)
    rb = jnp.sin(jnp.pi * n[None, :] * r[:, None] / _R_CUT) / r[:, None]
    rb = rb * fc[:, None]
    dx, dy, dz = d[:, 0], d[:, 1], d[:, 2]
    c0 = 0.28209479177387814
    c1 = 0.4886025119029199
    c2a = 1.0925484305920792
    c2b = 0.31539156525252005
    c2c = 0.5462742152960396
    ms = [
        jnp.full_like(dx, c0),
        c1 * dy, c1 * dz, c1 * dx,
        c2a * dx * dy,
        c2a * dy * dz,
        c2b * (3.0 * dz * dz - 1.0),
        c2a * dx * dz,
        c2c * (dx * dx - dy * dy),
    ]
    out_ref[...] = jnp.concatenate([m[:, None] * rb for m in ms], axis=-1)


def kernel(positions, edge_index, species):
    centers = edge_index[0].astype(jnp.int32)
    neighbors = edge_index[1].astype(jnp.int32)
    pc = positions[centers]
    pn = positions[neighbors]
    feat = pl.pallas_call(
        _expand_kernel,
        grid=(_E // _BLK,),
        in_specs=[
            pl.BlockSpec((_BLK, 3), lambda i: (i, 0)),
            pl.BlockSpec((_BLK, 3), lambda i: (i, 0)),
        ],
        out_specs=pl.BlockSpec((_BLK, _W), lambda i: (i, 0)),
        out_shape=jax.ShapeDtypeStruct((_E, _W), jnp.float32),
    )(pc, pn)
    seg = centers * _S + species[neighbors].astype(jnp.int32)
    summed = jax.ops.segment_sum(feat, seg, num_segments=_N * _S)
    outs = []
    off = 0
    for l in range(_L_MAX + 1):
        w = (2 * l + 1) * _N_MAX
        outs.append(summed[:, off:off + w].reshape(_N, _S * w))
        off += w
    return jnp.concatenate(outs, axis=-1)
